# Initial kernel scaffold; baseline (speedup 1.0000x reference)
import jax
import jax.numpy as jnp
from jax.experimental import pallas as pl


def kernel(X, labels):
    raise NotImplementedError("write your pallas kernel here")


# speedup vs baseline: 6.0557x; 6.0557x over previous
import jax
import jax.numpy as jnp
from jax import lax
from jax.experimental import pallas as pl
from jax.experimental.pallas import tpu as pltpu
from jax.experimental.pallas import tpu_sc as plsc

NUM_CLASSES = 10000
N_ROWS = 320000
D_COLS = 128
NC = 2
NS = 16
BLK = 512
NBLK = N_ROWS // BLK
BLKS_PER_TILE = NBLK // NS
CPT = NUM_CLASSES // NS
HALF = D_COLS // NC


def _sc_body(x_hbm, lab_hbm, numw_hbm, miu_hbm, std_hbm,
             miu_sh, std_sh, num_sh, xbuf, idxb, onesb, zbuf):
    cid = lax.axis_index("c")
    sid = lax.axis_index("s")
    c0 = cid * HALF

    zeros16 = jnp.zeros((16,), jnp.float32)
    ones16 = jnp.ones((16,), jnp.float32)

    @pl.loop(0, 128)
    def _(i):
        @pl.when(i < 64)
        def _():
            for c4 in range(HALF // 16):
                zbuf[i, pl.ds(c4 * 16, 16)] = zeros16
        onesb[i, pl.ds(0, 16)] = ones16

    base = sid * CPT
    for off, n in ((0, 64), (64, 64), (128, 64), (192, 64), (256, 64),
                   (320, 64), (384, 64), (448, 64), (512, 64), (576, 49)):
        pltpu.sync_copy(zbuf.at[pl.ds(0, n), :],
                        miu_sh.at[pl.ds(base + off, n), :])
        pltpu.sync_copy(zbuf.at[pl.ds(0, n), :],
                        std_sh.at[pl.ds(base + off, n), :])

    @pl.when(cid == 0)
    def _():
        for j in range(10):
            n = 625 - 9 * 64 if j == 9 else 64
            pltpu.sync_copy(zbuf.at[pl.ds(0, n), pl.ds(0, 16)],
                            num_sh.at[pl.ds(base + j * 64, n), :])

    plsc.subcore_barrier()

    def process(b):
        r0 = b * BLK
        pltpu.sync_copy(x_hbm.at[pl.ds(r0, BLK), pl.ds(c0, HALF)], xbuf)
        pltpu.sync_copy(lab_hbm.at[pl.ds(b * (BLK // 128), BLK // 128), :],
                        idxb)

        for j in range(BLK // 128):
            rows = pl.ds(j * 128, 128)
            idx = idxb.at[j]
            pltpu.sync_copy(xbuf.at[rows, :], miu_sh.at[idx], add=True)

            @pl.when(cid == 0)
            def _():
                pltpu.sync_copy(onesb, num_sh.at[idx], add=True)

        @pl.loop(0, BLK)
        def _(i):
            for c4 in range(HALF // 16):
                v = xbuf[i, pl.ds(c4 * 16, 16)]
                xbuf[i, pl.ds(c4 * 16, 16)] = v * v

        for j in range(BLK // 128):
            rows = pl.ds(j * 128, 128)
            idx = idxb.at[j]
            pltpu.sync_copy(xbuf.at[rows, :], std_sh.at[idx], add=True)

    @pl.loop(0, BLKS_PER_TILE)
    def _(k):
        process(sid + k * NS)

    @pl.when(sid == 0)
    def _():
        process(NBLK - 1)

    plsc.subcore_barrier()

    rows = pl.ds(base, CPT)
    pltpu.sync_copy(miu_sh.at[rows, :], miu_hbm.at[rows, pl.ds(c0, HALF)])
    pltpu.sync_copy(std_sh.at[rows, :], std_hbm.at[rows, pl.ds(c0, HALF)])

    @pl.when(cid == 0)
    def _():
        pltpu.sync_copy(num_sh.at[rows, :], numw_hbm.at[rows, :])


@jax.jit
def _tracker(X, labels2d):
    mesh = plsc.VectorSubcoreMesh(core_axis_name="c", subcore_axis_name="s")
    f = pl.kernel(
        _sc_body,
        compiler_params=pltpu.CompilerParams(use_tc_tiling_on_sc=False),
        out_type=(
            jax.ShapeDtypeStruct((NUM_CLASSES, 16), jnp.float32),
            jax.ShapeDtypeStruct((NUM_CLASSES, D_COLS), jnp.float32),
            jax.ShapeDtypeStruct((NUM_CLASSES, D_COLS), jnp.float32),
        ),
        mesh=mesh,
        scratch_types=[
            pltpu.VMEM_SHARED((NUM_CLASSES, HALF), jnp.float32),
            pltpu.VMEM_SHARED((NUM_CLASSES, HALF), jnp.float32),
            pltpu.VMEM_SHARED((NUM_CLASSES, 16), jnp.float32),
            pltpu.VMEM((BLK, HALF), jnp.float32),
            pltpu.VMEM((BLK // 128, 128), jnp.int32),
            pltpu.VMEM((128, 16), jnp.float32),
            pltpu.VMEM((64, HALF), jnp.float32),
        ],
    )
    return f(X, labels2d)


def kernel(X, labels):
    labels2d = labels.astype(jnp.int32).reshape(N_ROWS // 128, 128)
    numw, miu, std = _tracker(X, labels2d)
    return (numw[:, :1], miu, std)


# speedup vs baseline: 9.2608x; 1.5293x over previous
import jax
import jax.numpy as jnp
from jax import lax
from jax.experimental import pallas as pl
from jax.experimental.pallas import tpu as pltpu
from jax.experimental.pallas import tpu_sc as plsc

NUM_CLASSES = 10000
N_ROWS = 320000
D_COLS = 128
NC = 2
NS = 16
BLK = 128
NBLK = N_ROWS // BLK
BLKS_PER_TILE = NBLK // NS
EXTRA = NBLK - BLKS_PER_TILE * NS
CPT = NUM_CLASSES // NS
HALF = D_COLS // NC


def _sc_body(x_hbm, lab_hbm, numw_hbm, miu_hbm, std_hbm,
             miu_sh, std_sh, num_sh, xa, xb_, sqa, sqb_, idxb, onesb, zbuf,
             isem_a, isem_b, ssem_a, ssem_b):
    cid = lax.axis_index("c")
    sid = lax.axis_index("s")
    c0 = cid * HALF
    bufs = ((xa, sqa, isem_a, ssem_a), (xb_, sqb_, isem_b, ssem_b))

    def xslice(b):
        return x_hbm.at[pl.ds(b * BLK, BLK), pl.ds(c0, HALF)]

    for par in range(2):
        xv, _, isem, _ = bufs[par]
        pltpu.async_copy(xslice(sid + par * NS), xv, isem)
        pltpu.async_copy(lab_hbm.at[sid + par * NS], idxb.at[par], isem)

    zeros16 = jnp.zeros((16,), jnp.float32)
    ones16 = jnp.ones((16,), jnp.float32)

    @pl.loop(0, 128)
    def _(i):
        @pl.when(i < 64)
        def _():
            for c4 in range(HALF // 16):
                zbuf[i, pl.ds(c4 * 16, 16)] = zeros16
        onesb[i, pl.ds(0, 16)] = ones16

    base = sid * CPT
    for off, n in ((0, 64), (64, 64), (128, 64), (192, 64), (256, 64),
                   (320, 64), (384, 64), (448, 64), (512, 64), (576, 49)):
        pltpu.sync_copy(zbuf.at[pl.ds(0, n), :],
                        miu_sh.at[pl.ds(base + off, n), :])
        pltpu.sync_copy(zbuf.at[pl.ds(0, n), :],
                        std_sh.at[pl.ds(base + off, n), :])

    @pl.when(cid == 0)
    def _():
        for j in range(10):
            n = 625 - 9 * 64 if j == 9 else 64
            pltpu.sync_copy(zbuf.at[pl.ds(0, n), pl.ds(0, 16)],
                            num_sh.at[pl.ds(base + j * 64, n), :])

    plsc.subcore_barrier()

    def square(src, dst):
        @pl.loop(0, BLK)
        def _(i):
            for c4 in range(HALF // 16):
                v = src[i, pl.ds(c4 * 16, 16)]
                dst[i, pl.ds(c4 * 16, 16)] = v * v

    @pl.loop(0, BLKS_PER_TILE, step=2)
    def _(k):
        for par in range(2):
            kk = k + par
            xv, sqv, isem, ssem = bufs[par]
            idx = idxb.at[par]
            pltpu.make_async_copy(xslice(sid), xv, isem).wait()
            pltpu.make_async_copy(lab_hbm.at[sid], idx, isem).wait()
            cp_miu = pltpu.async_copy(xv, miu_sh.at[idx], ssem, add=True)

            @pl.when(cid == 0)
            def _():
                pltpu.async_copy(onesb, num_sh.at[idx], ssem, add=True)

            square(xv, sqv)
            cp_std = pltpu.async_copy(sqv, std_sh.at[idx], ssem, add=True)
            cp_miu.wait()
            cp_std.wait()

            @pl.when(cid == 0)
            def _():
                pltpu.make_async_copy(onesb, num_sh.at[idx], ssem).wait()

            @pl.when(kk + 2 < BLKS_PER_TILE)
            def _():
                b_next = sid + (kk + 2) * NS
                pltpu.async_copy(xslice(b_next), xv, isem)
                pltpu.async_copy(lab_hbm.at[b_next], idx, isem)

    @pl.when(sid < EXTRA)
    def _():
        b = BLKS_PER_TILE * NS + sid
        xv, sqv, _, _ = bufs[0]
        idx = idxb.at[0]
        pltpu.sync_copy(xslice(b), xv)
        pltpu.sync_copy(lab_hbm.at[b], idx)
        pltpu.sync_copy(xv, miu_sh.at[idx], add=True)

        @pl.when(cid == 0)
        def _():
            pltpu.sync_copy(onesb, num_sh.at[idx], add=True)

        square(xv, sqv)
        pltpu.sync_copy(sqv, std_sh.at[idx], add=True)

    plsc.subcore_barrier()

    rows = pl.ds(base, CPT)
    pltpu.sync_copy(miu_sh.at[rows, :], miu_hbm.at[rows, pl.ds(c0, HALF)])
    pltpu.sync_copy(std_sh.at[rows, :], std_hbm.at[rows, pl.ds(c0, HALF)])

    @pl.when(cid == 0)
    def _():
        pltpu.sync_copy(num_sh.at[rows, :], numw_hbm.at[rows, :])


@jax.jit
def _tracker(X, labels2d):
    mesh = plsc.VectorSubcoreMesh(core_axis_name="c", subcore_axis_name="s")
    f = pl.kernel(
        _sc_body,
        compiler_params=pltpu.CompilerParams(use_tc_tiling_on_sc=False),
        out_type=(
            jax.ShapeDtypeStruct((NUM_CLASSES, 16), jnp.float32),
            jax.ShapeDtypeStruct((NUM_CLASSES, D_COLS), jnp.float32),
            jax.ShapeDtypeStruct((NUM_CLASSES, D_COLS), jnp.float32),
        ),
        mesh=mesh,
        scratch_types=[
            pltpu.VMEM_SHARED((NUM_CLASSES, HALF), jnp.float32),
            pltpu.VMEM_SHARED((NUM_CLASSES, HALF), jnp.float32),
            pltpu.VMEM_SHARED((NUM_CLASSES, 16), jnp.float32),
            pltpu.VMEM((BLK, HALF), jnp.float32),
            pltpu.VMEM((BLK, HALF), jnp.float32),
            pltpu.VMEM((BLK, HALF), jnp.float32),
            pltpu.VMEM((BLK, HALF), jnp.float32),
            pltpu.VMEM((2, 128), jnp.int32),
            pltpu.VMEM((128, 16), jnp.float32),
            pltpu.VMEM((64, HALF), jnp.float32),
            pltpu.SemaphoreType.DMA,
            pltpu.SemaphoreType.DMA,
            pltpu.SemaphoreType.DMA,
            pltpu.SemaphoreType.DMA,
        ],
    )
    return f(X, labels2d)


def kernel(X, labels):
    labels2d = labels.astype(jnp.int32).reshape(N_ROWS // 128, 128)
    numw, miu, std = _tracker(X, labels2d)
    return (numw[:, :1], miu, std)
